# Initial kernel scaffold; baseline (speedup 1.0000x reference)
import jax
import jax.numpy as jnp
from jax.experimental import pallas as pl


def kernel(edge_index, feat, shuf_feat, temp, W1, b1, alpha, beta, Wb, bb):
    raise NotImplementedError("write your pallas kernel here")


# speedup vs baseline: 7.0050x; 7.0050x over previous
import functools
import math

import jax
import jax.numpy as jnp
import numpy as np
from jax import lax
from jax.experimental import pallas as pl
from jax.experimental.pallas import tpu as pltpu
from jax.experimental.pallas import tpu_sc as plsc

N = 10000
E = 320000
D = 128
DH = 32
NH = D // DH
KCH = 10
ROWS = 10240
TROWS = 640
NS = 16
ECH = 160
EPT = ECH * 128
ETOT = EPT * NS
DUMMY = 10200


def _interp_matrix():
    def cheb(i, x):
        if i == 0:
            return 1.0
        t0, t1 = 1.0, x
        for _ in range(2, i + 1):
            t0, t1 = t1, 2.0 * x * t1 - t0
        return t1
    xs = [math.cos((KCH - j + 0.5) * math.pi / (KCH + 1)) for j in range(KCH + 1)]
    M = np.zeros((KCH + 1, KCH + 1), dtype=np.float32)
    for i in range(KCH + 1):
        for j in range(KCH + 1):
            M[i, j] = cheb(i, xs[j])
    return M


_INTERP_F32 = _interp_matrix()


def _sc_body(pkT, Z, coe16, low, high, tbuf, gbuf,
             srcb, dstb, rowA, rowB, bufL, bufH, zerob,
             dinvb, coev,
             acc, sg0, sg1, sd):
    c = lax.axis_index("c")
    s = lax.axis_index("s")
    base_e = s * ECH
    base_r = s * TROWS

    def _zrow(i, carry):
        for g in range(DH // 16):
            zerob[i, pl.ds(16 * g, 16)] = jnp.zeros((16,), jnp.float32)
            bufL[i, pl.ds(16 * g, 16)] = jnp.ones((16,), jnp.float32)
        return carry
    lax.fori_loop(0, 128, _zrow, 0)
    for j in range(5):
        pltpu.sync_copy(zerob, acc.at[pl.ds(base_r + 128 * j, 128)])

    pltpu.sync_copy(pkT.at[pl.ds(base_e, ECH)], srcb)

    def _unpack(w, carry):
        for g in range(8):
            ix = pl.ds(16 * g, 16)
            pk = srcb[w, ix]
            dstb[w, ix] = pk & 16383
            srcb[w, ix] = pk >> 14
        return carry
    lax.fori_loop(0, ECH, _unpack, 0)

    pltpu.sync_copy(coe16, coev.at[pl.ds(0, 16)])
    coev[pl.ds(16, 16)] = jnp.zeros((16,), jnp.float32)

    plsc.subcore_barrier()

    def _deg(i, carry):
        for b in range(8):
            pltpu.async_copy(bufL, acc.at[dstb.at[8 * i + b]], sd, add=True)
        for b in range(8):
            pltpu.make_async_copy(bufL, acc.at[dstb.at[8 * i + b]], sd).wait()
        return carry
    lax.fori_loop(0, ECH // 8, _deg, 0)
    plsc.subcore_barrier()

    for j in range(5):
        r0 = base_r + 128 * j
        pltpu.sync_copy(acc.at[pl.ds(r0, 128)], rowA)

        def _newton(w, carry):
            for g in range(DH // 16):
                ix = pl.ds(16 * g, 16)
                d = rowA[w, ix]
                bits = lax.bitcast_convert_type(d, jnp.int32)
                y = lax.bitcast_convert_type(
                    jnp.full((16,), 0x5F3759DF, jnp.int32) - (bits >> 1),
                    jnp.float32)
                for _ in range(4):
                    y = y * (1.5 - 0.5 * d * y * y)
                dinvb[128 * j + w, ix] = jnp.where(d > 0.5, y, 0.0)
            return carry
        lax.fori_loop(0, 128, _newton, 0)
        pltpu.sync_copy(zerob, acc.at[pl.ds(r0, 128)])

    c0 = coev[pl.ds(0, 16)][0] * 0.5
    for h in range(NH):
        for j in range(5):
            r0 = base_r + 128 * j
            pltpu.sync_copy(Z.at[c].at[h].at[pl.ds(r0, 128)], rowA)

            def _irow(w, carry):
                for g in range(DH // 16):
                    ix = pl.ds(16 * g, 16)
                    zv = rowA[w, ix]
                    rowB[w, ix] = dinvb[128 * j + w, ix] * zv
                    bufL[w, ix] = c0 * zv
                return carry
            lax.fori_loop(0, 128, _irow, 0)
            pltpu.sync_copy(rowA, tbuf.at[c].at[h].at[0].at[pl.ds(r0, 128)])
            pltpu.sync_copy(zerob, tbuf.at[c].at[h].at[1].at[pl.ds(r0, 128)])
            pltpu.sync_copy(rowB, gbuf.at[c].at[h].at[pl.ds(r0, 128)])
            pltpu.sync_copy(bufL, low.at[c].at[h].at[pl.ds(r0, 128)])
            pltpu.sync_copy(bufL, high.at[c].at[h].at[pl.ds(r0, 128)])
    plsc.subcore_barrier()

    coevec = coev[pl.ds(0, 16)]

    def _round(r, carry):
        rb16 = jnp.full((16,), r, jnp.int32)
        cr = coevec[rb16]
        sgn = 1.0 - 2.0 * (r % 2).astype(jnp.float32)
        scale = jnp.where(r == 1, 1.0, 2.0)
        msub = jnp.where(r == 1, 0.0, 1.0)
        p = r % 2

        for h in range(NH):
            gh = gbuf.at[c].at[h]
            pltpu.async_copy(gh.at[srcb.at[0]], rowA, sg0)
            pltpu.async_copy(gh.at[srcb.at[1]], rowB, sg1)

            def _edge(i, carry2):
                for b in range(2):
                    k = 2 * i + b
                    rb = rowA if b == 0 else rowB
                    sg = sg0 if b == 0 else sg1
                    pltpu.make_async_copy(gh.at[srcb.at[k]], rb, sg).wait()
                    pltpu.sync_copy(rb, acc.at[dstb.at[k]], add=True)

                    @pl.when(k + 2 < ECH)
                    def _():
                        pltpu.async_copy(gh.at[srcb.at[k + 2]], rb, sg)
                return carry2
            lax.fori_loop(0, ECH // 2, _edge, 0)
            plsc.subcore_barrier()

            for j in range(5):
                r0 = base_r + 128 * j
                pltpu.sync_copy(acc.at[pl.ds(r0, 128)], rowA)
                pltpu.sync_copy(tbuf.at[c].at[h].at[p].at[pl.ds(r0, 128)], rowB)
                pltpu.sync_copy(low.at[c].at[h].at[pl.ds(r0, 128)], bufL)
                pltpu.sync_copy(high.at[c].at[h].at[pl.ds(r0, 128)], bufH)

                def _drow(w, carry2):
                    for g in range(DH // 16):
                        ix = pl.ds(16 * g, 16)
                        dv = dinvb[128 * j + w, ix]
                        pv = rowA[w, ix]
                        t2 = rowB[w, ix]
                        tn = (scale * dv) * pv - msub * t2
                        rowB[w, ix] = tn
                        rowA[w, ix] = dv * tn
                        bufL[w, ix] = bufL[w, ix] + cr * tn
                        bufH[w, ix] = bufH[w, ix] + (sgn * cr) * tn
                    return carry2
                lax.fori_loop(0, 128, _drow, 0)
                pltpu.sync_copy(rowB, tbuf.at[c].at[h].at[p].at[pl.ds(r0, 128)])
                pltpu.sync_copy(rowA, gbuf.at[c].at[h].at[pl.ds(r0, 128)])
                pltpu.sync_copy(bufL, low.at[c].at[h].at[pl.ds(r0, 128)])
                pltpu.sync_copy(bufH, high.at[c].at[h].at[pl.ds(r0, 128)])
                pltpu.sync_copy(zerob, acc.at[pl.ds(r0, 128)])
            plsc.subcore_barrier()
        return carry
    lax.fori_loop(1, KCH + 1, _round, 0)


_sc_cheb = functools.partial(
    pl.kernel,
    out_type=[
        jax.ShapeDtypeStruct((2, NH, ROWS, DH), jnp.float32),
        jax.ShapeDtypeStruct((2, NH, ROWS, DH), jnp.float32),
        jax.ShapeDtypeStruct((2, NH, 2, ROWS, DH), jnp.float32),
        jax.ShapeDtypeStruct((2, NH, ROWS, DH), jnp.float32),
    ],
    mesh=plsc.VectorSubcoreMesh(core_axis_name="c", subcore_axis_name="s"),
    compiler_params=pltpu.CompilerParams(use_tc_tiling_on_sc=False),
    scratch_types=[
        pltpu.VMEM((ECH, 128), jnp.int32),
        pltpu.VMEM((ECH, 128), jnp.int32),
        pltpu.VMEM((128, DH), jnp.float32),
        pltpu.VMEM((128, DH), jnp.float32),
        pltpu.VMEM((128, DH), jnp.float32),
        pltpu.VMEM((128, DH), jnp.float32),
        pltpu.VMEM((128, DH), jnp.float32),
        pltpu.VMEM((TROWS, DH), jnp.float32),
        pltpu.VMEM((32,), jnp.float32),
        pltpu.VMEM_SHARED((ROWS, DH), jnp.float32),
        pltpu.SemaphoreType.DMA,
        pltpu.SemaphoreType.DMA,
        pltpu.SemaphoreType.DMA,
    ],
)(_sc_body)


def _proj_body(x_ref, w_ref, z_ref):
    z_ref[0, 0] = lax.dot_general(x_ref[0], w_ref[...],
                                  (((1,), (1,)), ((), ())),
                                  precision=lax.Precision.HIGHEST,
                                  preferred_element_type=jnp.float32)


def _cat(ref, ci):
    return jnp.concatenate([ref[ci, h] for h in range(NH)], axis=-1)


def _b1_body(low_ref, high_ref, b1_ref, ab_ref, out_ref):
    i = pl.program_id(0)
    b = b1_ref[0]
    h1 = jnp.maximum(_cat(high_ref, 0) + b, 0.0)
    h2 = jnp.maximum(_cat(low_ref, 0) + b, 0.0)
    h = ab_ref[0, 0] * h1 + ab_ref[0, 1] * h2

    @pl.when(i == 0)
    def _():
        out_ref[...] = jnp.zeros_like(out_ref)
    out_ref[...] += jnp.sum(h, axis=0, keepdims=True)


def _b2_body(low_ref, high_ref, b1_ref, hsum_ref, wb_ref, bb_ref, out_ref):
    cm = jnp.maximum(hsum_ref[...] * jnp.float32(1.0 / N), 0.0)
    wrow = lax.dot_general(cm, wb_ref[0], (((1,), (1,)), ((), ())),
                           precision=lax.Precision.HIGHEST,
                           preferred_element_type=jnp.float32)
    b = b1_ref[0]
    bbv = bb_ref[0, 0]
    h2 = jnp.maximum(_cat(low_ref, 0) + b, 0.0)
    h1 = jnp.maximum(_cat(high_ref, 0) + b, 0.0)
    h4 = jnp.maximum(_cat(low_ref, 1) + b, 0.0)
    h3 = jnp.maximum(_cat(high_ref, 1) + b, 0.0)

    def dots(hm):
        r = lax.dot_general(hm, wrow, (((1,), (1,)), ((), ())),
                            precision=lax.Precision.HIGHEST,
                            preferred_element_type=jnp.float32)
        return r[:, 0] + bbv
    out_ref[0, :] = dots(h2)
    out_ref[1, :] = dots(h1)
    out_ref[2, :] = dots(h4)
    out_ref[3, :] = dots(h3)


def kernel(edge_index, feat, shuf_feat, temp, W1, b1, alpha, beta, Wb, bb):
    f32 = jnp.float32
    pad_rows = jnp.zeros((ROWS - N, D), f32)
    Xp = jnp.stack([jnp.concatenate([feat, pad_rows]),
                    jnp.concatenate([shuf_feat, pad_rows])])
    pad_e = jnp.full((ETOT - E,), DUMMY, jnp.int32)
    srcp = jnp.concatenate([edge_index[0], pad_e])
    dstp = jnp.concatenate([edge_index[1], pad_e])
    pkT = ((srcp << 14) | dstp).reshape(NS * ECH, 128)
    coe = (2.0 / (KCH + 1)) * (jnp.asarray(_INTERP_F32) @ jax.nn.relu(temp))
    coe16 = jnp.concatenate([coe.astype(f32), jnp.zeros((16 - (KCH + 1),), f32)])

    Z = pl.pallas_call(
        _proj_body,
        grid=(2, NH, ROWS // 256),
        in_specs=[pl.BlockSpec((1, 256, D), lambda i, h, j: (i, j, 0)),
                  pl.BlockSpec((DH, D), lambda i, h, j: (h, 0))],
        out_specs=pl.BlockSpec((1, 1, 256, DH), lambda i, h, j: (i, h, j, 0)),
        out_shape=jax.ShapeDtypeStruct((2, NH, ROWS, DH), f32),
    )(Xp, W1)

    low, high, _t, _g = _sc_cheb(pkT, Z, coe16)

    ab = jnp.stack([alpha.astype(f32), beta.astype(f32)]).reshape(1, 2)
    hsum = pl.pallas_call(
        _b1_body,
        grid=(25,),
        in_specs=[pl.BlockSpec((1, NH, 400, DH), lambda i: (0, 0, i, 0)),
                  pl.BlockSpec((1, NH, 400, DH), lambda i: (0, 0, i, 0)),
                  pl.BlockSpec((1, D), lambda i: (0, 0)),
                  pl.BlockSpec((1, 2), lambda i: (0, 0))],
        out_specs=pl.BlockSpec((1, D), lambda i: (0, 0)),
        out_shape=jax.ShapeDtypeStruct((1, D), f32),
    )(low, high, b1.reshape(1, D), ab)

    out4 = pl.pallas_call(
        _b2_body,
        grid=(ROWS // 512,),
        in_specs=[pl.BlockSpec((2, NH, 512, DH), lambda i: (0, 0, i, 0)),
                  pl.BlockSpec((2, NH, 512, DH), lambda i: (0, 0, i, 0)),
                  pl.BlockSpec((1, D), lambda i: (0, 0)),
                  pl.BlockSpec((1, D), lambda i: (0, 0)),
                  pl.BlockSpec((1, D, D), lambda i: (0, 0, 0)),
                  pl.BlockSpec((1, 1), lambda i: (0, 0))],
        out_specs=pl.BlockSpec((4, 512), lambda i: (0, i)),
        out_shape=jax.ShapeDtypeStruct((4, ROWS), f32),
    )(low, high, b1.reshape(1, D), hsum, Wb, bb.reshape(1, 1))

    return jnp.concatenate([out4[0, :N], out4[1, :N], out4[2, :N], out4[3, :N]])


# speedup vs baseline: 7.3164x; 1.0444x over previous
import functools
import math

import jax
import jax.numpy as jnp
import numpy as np
from jax import lax
from jax.experimental import pallas as pl
from jax.experimental.pallas import tpu as pltpu
from jax.experimental.pallas import tpu_sc as plsc

N = 10000
E = 320000
D = 128
DH = 32
NH = D // DH
KCH = 10
ROWS = 10240
TROWS = 640
NS = 16
ECH = 160
EPT = ECH * 128
ETOT = EPT * NS
DUMMY = 10200


def _interp_matrix():
    def cheb(i, x):
        if i == 0:
            return 1.0
        t0, t1 = 1.0, x
        for _ in range(2, i + 1):
            t0, t1 = t1, 2.0 * x * t1 - t0
        return t1
    xs = [math.cos((KCH - j + 0.5) * math.pi / (KCH + 1)) for j in range(KCH + 1)]
    M = np.zeros((KCH + 1, KCH + 1), dtype=np.float32)
    for i in range(KCH + 1):
        for j in range(KCH + 1):
            M[i, j] = cheb(i, xs[j])
    return M


_INTERP_F32 = _interp_matrix()


def _sc_body(pkT, Z, coe16, low, high, tbuf, gbuf,
             srcb, dstb, rowA, rowB, bufL, bufH, zerob,
             dinvb, coev,
             acc, sg0, sg1, sg2, sg3, ss0, ss1, ss2, ss3, sd):
    c = lax.axis_index("c")
    s = lax.axis_index("s")
    base_e = s * ECH
    base_r = s * TROWS

    def _zrow(i, carry):
        for g in range(DH // 16):
            zerob[i, pl.ds(16 * g, 16)] = jnp.zeros((16,), jnp.float32)
            bufL[i, pl.ds(16 * g, 16)] = jnp.ones((16,), jnp.float32)
        return carry
    lax.fori_loop(0, 128, _zrow, 0)
    for j in range(5):
        pltpu.sync_copy(zerob, acc.at[pl.ds(base_r + 128 * j, 128)])

    pltpu.sync_copy(pkT.at[pl.ds(base_e, ECH)], srcb)

    def _unpack(w, carry):
        for g in range(8):
            ix = pl.ds(16 * g, 16)
            pk = srcb[w, ix]
            dstb[w, ix] = pk & 16383
            srcb[w, ix] = pk >> 14
        return carry
    lax.fori_loop(0, ECH, _unpack, 0)

    pltpu.sync_copy(coe16, coev.at[pl.ds(0, 16)])
    coev[pl.ds(16, 16)] = jnp.zeros((16,), jnp.float32)

    plsc.subcore_barrier()

    def _deg(i, carry):
        for b in range(8):
            pltpu.async_copy(bufL, acc.at[dstb.at[8 * i + b]], sd, add=True)
        for b in range(8):
            pltpu.make_async_copy(bufL, acc.at[dstb.at[8 * i + b]], sd).wait()
        return carry
    lax.fori_loop(0, ECH // 8, _deg, 0)
    plsc.subcore_barrier()

    for j in range(5):
        r0 = base_r + 128 * j
        pltpu.sync_copy(acc.at[pl.ds(r0, 128)], rowA)

        def _newton(w, carry):
            for g in range(DH // 16):
                ix = pl.ds(16 * g, 16)
                d = rowA[w, ix]
                bits = lax.bitcast_convert_type(d, jnp.int32)
                y = lax.bitcast_convert_type(
                    jnp.full((16,), 0x5F3759DF, jnp.int32) - (bits >> 1),
                    jnp.float32)
                for _ in range(4):
                    y = y * (1.5 - 0.5 * d * y * y)
                dinvb[128 * j + w, ix] = jnp.where(d > 0.5, y, 0.0)
            return carry
        lax.fori_loop(0, 128, _newton, 0)
        pltpu.sync_copy(zerob, acc.at[pl.ds(r0, 128)])

    c0 = coev[pl.ds(0, 16)][0] * 0.5
    for h in range(NH):
        for j in range(5):
            r0 = base_r + 128 * j
            pltpu.sync_copy(Z.at[c].at[h].at[pl.ds(r0, 128)], rowA)

            def _irow(w, carry):
                for g in range(DH // 16):
                    ix = pl.ds(16 * g, 16)
                    zv = rowA[w, ix]
                    rowB[w, ix] = dinvb[128 * j + w, ix] * zv
                    bufL[w, ix] = c0 * zv
                return carry
            lax.fori_loop(0, 128, _irow, 0)
            pltpu.sync_copy(rowA, tbuf.at[c].at[h].at[0].at[pl.ds(r0, 128)])
            pltpu.sync_copy(zerob, tbuf.at[c].at[h].at[1].at[pl.ds(r0, 128)])
            pltpu.sync_copy(rowB, gbuf.at[c].at[h].at[pl.ds(r0, 128)])
            pltpu.sync_copy(bufL, low.at[c].at[h].at[pl.ds(r0, 128)])
            pltpu.sync_copy(bufL, high.at[c].at[h].at[pl.ds(r0, 128)])
    plsc.subcore_barrier()

    coevec = coev[pl.ds(0, 16)]

    def _round(r, carry):
        rb16 = jnp.full((16,), r, jnp.int32)
        cr = coevec[rb16]
        sgn = 1.0 - 2.0 * (r % 2).astype(jnp.float32)
        scale = jnp.where(r == 1, 1.0, 2.0)
        msub = jnp.where(r == 1, 0.0, 1.0)
        p = r % 2

        for h in range(NH):
            gh = gbuf.at[c].at[h]
            rbs = (rowA, rowB, bufL, bufH)
            sgs = (sg0, sg1, sg2, sg3)
            sss = (ss0, ss1, ss2, ss3)
            pltpu.async_copy(gh.at[srcb.at[0]], rbs[0], sgs[0])
            pltpu.async_copy(gh.at[srcb.at[1]], rbs[1], sgs[1])

            def _edge(i, carry2):
                for b in range(4):
                    k = 4 * i + b
                    b2 = (b + 2) % 4
                    pltpu.make_async_copy(gh.at[srcb.at[k]], rbs[b], sgs[b]).wait()
                    pltpu.async_copy(rbs[b], acc.at[dstb.at[k]], sss[b], add=True)

                    @pl.when(k + 2 < ECH)
                    def _():
                        @pl.when(k >= 2)
                        def _():
                            pltpu.make_async_copy(
                                rbs[b2], acc.at[dstb.at[k - 2]], sss[b2]).wait()
                        pltpu.async_copy(gh.at[srcb.at[k + 2]], rbs[b2], sgs[b2])
                return carry2
            lax.fori_loop(0, ECH // 4, _edge, 0)
            for b in range(4):
                pltpu.make_async_copy(
                    rbs[b], acc.at[dstb.at[ECH - 4 + b]], sss[b]).wait()
            plsc.subcore_barrier()

            for j in range(5):
                r0 = base_r + 128 * j
                pltpu.sync_copy(acc.at[pl.ds(r0, 128)], rowA)
                pltpu.sync_copy(tbuf.at[c].at[h].at[p].at[pl.ds(r0, 128)], rowB)
                pltpu.sync_copy(low.at[c].at[h].at[pl.ds(r0, 128)], bufL)
                pltpu.sync_copy(high.at[c].at[h].at[pl.ds(r0, 128)], bufH)

                def _drow(w, carry2):
                    for g in range(DH // 16):
                        ix = pl.ds(16 * g, 16)
                        dv = dinvb[128 * j + w, ix]
                        pv = rowA[w, ix]
                        t2 = rowB[w, ix]
                        tn = (scale * dv) * pv - msub * t2
                        rowB[w, ix] = tn
                        rowA[w, ix] = dv * tn
                        bufL[w, ix] = bufL[w, ix] + cr * tn
                        bufH[w, ix] = bufH[w, ix] + (sgn * cr) * tn
                    return carry2
                lax.fori_loop(0, 128, _drow, 0)
                pltpu.sync_copy(rowB, tbuf.at[c].at[h].at[p].at[pl.ds(r0, 128)])
                pltpu.sync_copy(rowA, gbuf.at[c].at[h].at[pl.ds(r0, 128)])
                pltpu.sync_copy(bufL, low.at[c].at[h].at[pl.ds(r0, 128)])
                pltpu.sync_copy(bufH, high.at[c].at[h].at[pl.ds(r0, 128)])
                pltpu.sync_copy(zerob, acc.at[pl.ds(r0, 128)])
            plsc.subcore_barrier()
        return carry
    lax.fori_loop(1, KCH + 1, _round, 0)


_sc_cheb = functools.partial(
    pl.kernel,
    out_type=[
        jax.ShapeDtypeStruct((2, NH, ROWS, DH), jnp.float32),
        jax.ShapeDtypeStruct((2, NH, ROWS, DH), jnp.float32),
        jax.ShapeDtypeStruct((2, NH, 2, ROWS, DH), jnp.float32),
        jax.ShapeDtypeStruct((2, NH, ROWS, DH), jnp.float32),
    ],
    mesh=plsc.VectorSubcoreMesh(core_axis_name="c", subcore_axis_name="s"),
    compiler_params=pltpu.CompilerParams(use_tc_tiling_on_sc=False),
    scratch_types=[
        pltpu.VMEM((ECH, 128), jnp.int32),
        pltpu.VMEM((ECH, 128), jnp.int32),
        pltpu.VMEM((128, DH), jnp.float32),
        pltpu.VMEM((128, DH), jnp.float32),
        pltpu.VMEM((128, DH), jnp.float32),
        pltpu.VMEM((128, DH), jnp.float32),
        pltpu.VMEM((128, DH), jnp.float32),
        pltpu.VMEM((TROWS, DH), jnp.float32),
        pltpu.VMEM((32,), jnp.float32),
        pltpu.VMEM_SHARED((ROWS, DH), jnp.float32),
        pltpu.SemaphoreType.DMA,
        pltpu.SemaphoreType.DMA,
        pltpu.SemaphoreType.DMA,
        pltpu.SemaphoreType.DMA,
        pltpu.SemaphoreType.DMA,
        pltpu.SemaphoreType.DMA,
        pltpu.SemaphoreType.DMA,
        pltpu.SemaphoreType.DMA,
        pltpu.SemaphoreType.DMA,
    ],
)(_sc_body)


def _proj_body(x_ref, w_ref, z_ref):
    z_ref[0, 0] = lax.dot_general(x_ref[0], w_ref[...],
                                  (((1,), (1,)), ((), ())),
                                  precision=lax.Precision.HIGHEST,
                                  preferred_element_type=jnp.float32)


def _cat(ref, ci):
    return jnp.concatenate([ref[ci, h] for h in range(NH)], axis=-1)


def _b1_body(low_ref, high_ref, b1_ref, ab_ref, out_ref):
    i = pl.program_id(0)
    b = b1_ref[0]
    h1 = jnp.maximum(_cat(high_ref, 0) + b, 0.0)
    h2 = jnp.maximum(_cat(low_ref, 0) + b, 0.0)
    h = ab_ref[0, 0] * h1 + ab_ref[0, 1] * h2

    @pl.when(i == 0)
    def _():
        out_ref[...] = jnp.zeros_like(out_ref)
    out_ref[...] += jnp.sum(h, axis=0, keepdims=True)


def _b2_body(low_ref, high_ref, b1_ref, hsum_ref, wb_ref, bb_ref, out_ref):
    cm = jnp.maximum(hsum_ref[...] * jnp.float32(1.0 / N), 0.0)
    wrow = lax.dot_general(cm, wb_ref[0], (((1,), (1,)), ((), ())),
                           precision=lax.Precision.HIGHEST,
                           preferred_element_type=jnp.float32)
    b = b1_ref[0]
    bbv = bb_ref[0, 0]
    h2 = jnp.maximum(_cat(low_ref, 0) + b, 0.0)
    h1 = jnp.maximum(_cat(high_ref, 0) + b, 0.0)
    h4 = jnp.maximum(_cat(low_ref, 1) + b, 0.0)
    h3 = jnp.maximum(_cat(high_ref, 1) + b, 0.0)

    def dots(hm):
        r = lax.dot_general(hm, wrow, (((1,), (1,)), ((), ())),
                            precision=lax.Precision.HIGHEST,
                            preferred_element_type=jnp.float32)
        return r[:, 0] + bbv
    out_ref[0, :] = dots(h2)
    out_ref[1, :] = dots(h1)
    out_ref[2, :] = dots(h4)
    out_ref[3, :] = dots(h3)


def kernel(edge_index, feat, shuf_feat, temp, W1, b1, alpha, beta, Wb, bb):
    f32 = jnp.float32
    pad_rows = jnp.zeros((ROWS - N, D), f32)
    Xp = jnp.stack([jnp.concatenate([feat, pad_rows]),
                    jnp.concatenate([shuf_feat, pad_rows])])
    pad_e = jnp.full((ETOT - E,), DUMMY, jnp.int32)
    srcp = jnp.concatenate([edge_index[0], pad_e])
    dstp = jnp.concatenate([edge_index[1], pad_e])
    pkT = ((srcp << 14) | dstp).reshape(NS * ECH, 128)
    coe = (2.0 / (KCH + 1)) * (jnp.asarray(_INTERP_F32) @ jax.nn.relu(temp))
    coe16 = jnp.concatenate([coe.astype(f32), jnp.zeros((16 - (KCH + 1),), f32)])

    Z = pl.pallas_call(
        _proj_body,
        grid=(2, NH, ROWS // 256),
        in_specs=[pl.BlockSpec((1, 256, D), lambda i, h, j: (i, j, 0)),
                  pl.BlockSpec((DH, D), lambda i, h, j: (h, 0))],
        out_specs=pl.BlockSpec((1, 1, 256, DH), lambda i, h, j: (i, h, j, 0)),
        out_shape=jax.ShapeDtypeStruct((2, NH, ROWS, DH), f32),
    )(Xp, W1)

    low, high, _t, _g = _sc_cheb(pkT, Z, coe16)

    ab = jnp.stack([alpha.astype(f32), beta.astype(f32)]).reshape(1, 2)
    hsum = pl.pallas_call(
        _b1_body,
        grid=(25,),
        in_specs=[pl.BlockSpec((1, NH, 400, DH), lambda i: (0, 0, i, 0)),
                  pl.BlockSpec((1, NH, 400, DH), lambda i: (0, 0, i, 0)),
                  pl.BlockSpec((1, D), lambda i: (0, 0)),
                  pl.BlockSpec((1, 2), lambda i: (0, 0))],
        out_specs=pl.BlockSpec((1, D), lambda i: (0, 0)),
        out_shape=jax.ShapeDtypeStruct((1, D), f32),
    )(low, high, b1.reshape(1, D), ab)

    out4 = pl.pallas_call(
        _b2_body,
        grid=(ROWS // 512,),
        in_specs=[pl.BlockSpec((2, NH, 512, DH), lambda i: (0, 0, i, 0)),
                  pl.BlockSpec((2, NH, 512, DH), lambda i: (0, 0, i, 0)),
                  pl.BlockSpec((1, D), lambda i: (0, 0)),
                  pl.BlockSpec((1, D), lambda i: (0, 0)),
                  pl.BlockSpec((1, D, D), lambda i: (0, 0, 0)),
                  pl.BlockSpec((1, 1), lambda i: (0, 0))],
        out_specs=pl.BlockSpec((4, 512), lambda i: (0, i)),
        out_shape=jax.ShapeDtypeStruct((4, ROWS), f32),
    )(low, high, b1.reshape(1, D), hsum, Wb, bb.reshape(1, 1))

    return jnp.concatenate([out4[0, :N], out4[1, :N], out4[2, :N], out4[3, :N]])


# speedup vs baseline: 8.0421x; 1.0992x over previous
import functools
import math

import jax
import jax.numpy as jnp
import numpy as np
from jax import lax
from jax.experimental import pallas as pl
from jax.experimental.pallas import tpu as pltpu
from jax.experimental.pallas import tpu_sc as plsc

N = 10000
E = 320000
D = 128
DH = 32
NH = D // DH
KCH = 10
ROWS = 10240
TROWS = 640
NS = 16
ECH = 80
EPT = ECH * 256
ETOT = EPT * NS
DUMMY = 10200


def _interp_matrix():
    def cheb(i, x):
        if i == 0:
            return 1.0
        t0, t1 = 1.0, x
        for _ in range(2, i + 1):
            t0, t1 = t1, 2.0 * x * t1 - t0
        return t1
    xs = [math.cos((KCH - j + 0.5) * math.pi / (KCH + 1)) for j in range(KCH + 1)]
    M = np.zeros((KCH + 1, KCH + 1), dtype=np.float32)
    for i in range(KCH + 1):
        for j in range(KCH + 1):
            M[i, j] = cheb(i, xs[j])
    return M


_INTERP_F32 = _interp_matrix()


def _sc_body(pkT, Z, coe16, low, high, tbuf, gbuf,
             srcb, dstb, rowA, rowB, bufL, bufH, zerob,
             dinvb, coev,
             acc, sg0, sg1, sg2, sg3, ss0, ss1, ss2, ss3, sd):
    c = lax.axis_index("c")
    s = lax.axis_index("s")
    base_e = s * ECH
    base_r = s * TROWS

    def _zrow(i, carry):
        for g in range(DH // 16):
            zerob[i, pl.ds(16 * g, 16)] = jnp.zeros((16,), jnp.float32)
        return carry
    lax.fori_loop(0, 128, _zrow, 0)

    def _onerow(i, carry):
        for g in range(DH // 16):
            bufL[i, pl.ds(16 * g, 16)] = jnp.ones((16,), jnp.float32)
        return carry
    lax.fori_loop(0, 256, _onerow, 0)
    for j in range(5):
        pltpu.sync_copy(zerob, acc.at[pl.ds(base_r + 128 * j, 128)])

    pltpu.sync_copy(pkT.at[pl.ds(base_e, ECH)], srcb)

    def _unpack(w, carry):
        for g in range(16):
            ix = pl.ds(16 * g, 16)
            pk = srcb[w, ix]
            dstb[w, ix] = pk & 16383
            srcb[w, ix] = pk >> 14
        return carry
    lax.fori_loop(0, ECH, _unpack, 0)

    pltpu.sync_copy(coe16, coev.at[pl.ds(0, 16)])
    coev[pl.ds(16, 16)] = jnp.zeros((16,), jnp.float32)

    plsc.subcore_barrier()

    def _deg(i, carry):
        for b in range(8):
            pltpu.async_copy(bufL, acc.at[dstb.at[8 * i + b]], sd, add=True)
        for b in range(8):
            pltpu.make_async_copy(bufL, acc.at[dstb.at[8 * i + b]], sd).wait()
        return carry
    lax.fori_loop(0, ECH // 8, _deg, 0)
    plsc.subcore_barrier()

    for j in range(5):
        r0 = base_r + 128 * j
        pltpu.sync_copy(acc.at[pl.ds(r0, 128)], rowA.at[pl.ds(0, 128)])

        def _newton(w, carry):
            for g in range(DH // 16):
                ix = pl.ds(16 * g, 16)
                d = rowA[w, ix]
                bits = lax.bitcast_convert_type(d, jnp.int32)
                y = lax.bitcast_convert_type(
                    jnp.full((16,), 0x5F3759DF, jnp.int32) - (bits >> 1),
                    jnp.float32)
                for _ in range(4):
                    y = y * (1.5 - 0.5 * d * y * y)
                dinvb[128 * j + w, ix] = jnp.where(d > 0.5, y, 0.0)
            return carry
        lax.fori_loop(0, 128, _newton, 0)
        pltpu.sync_copy(zerob, acc.at[pl.ds(r0, 128)])

    c0 = coev[pl.ds(0, 16)][0] * 0.5
    for h in range(NH):
        for j in range(5):
            r0 = base_r + 128 * j
            pltpu.sync_copy(Z.at[c].at[h].at[pl.ds(r0, 128)], rowA.at[pl.ds(0, 128)])

            def _irow(w, carry):
                for g in range(DH // 16):
                    ix = pl.ds(16 * g, 16)
                    zv = rowA[w, ix]
                    rowB[w, ix] = dinvb[128 * j + w, ix] * zv
                    bufL[w, ix] = c0 * zv
                return carry
            lax.fori_loop(0, 128, _irow, 0)
            pltpu.sync_copy(rowA.at[pl.ds(0, 128)], tbuf.at[c].at[h].at[0].at[pl.ds(r0, 128)])
            pltpu.sync_copy(zerob, tbuf.at[c].at[h].at[1].at[pl.ds(r0, 128)])
            pltpu.sync_copy(rowB.at[pl.ds(0, 128)], gbuf.at[c].at[h].at[pl.ds(r0, 128)])
            pltpu.sync_copy(bufL.at[pl.ds(0, 128)], low.at[c].at[h].at[pl.ds(r0, 128)])
            pltpu.sync_copy(bufL.at[pl.ds(0, 128)], high.at[c].at[h].at[pl.ds(r0, 128)])
    plsc.subcore_barrier()

    coevec = coev[pl.ds(0, 16)]

    def _round(r, carry):
        rb16 = jnp.full((16,), r, jnp.int32)
        cr = coevec[rb16]
        sgn = 1.0 - 2.0 * (r % 2).astype(jnp.float32)
        scale = jnp.where(r == 1, 1.0, 2.0)
        msub = jnp.where(r == 1, 0.0, 1.0)
        p = r % 2

        for h in range(NH):
            gh = gbuf.at[c].at[h]
            rbs = (rowA, rowB, bufL, bufH)
            sgs = (sg0, sg1, sg2, sg3)
            sss = (ss0, ss1, ss2, ss3)
            pltpu.async_copy(gh.at[srcb.at[0]], rbs[0], sgs[0])
            pltpu.async_copy(gh.at[srcb.at[1]], rbs[1], sgs[1])

            def _edge(i, carry2):
                for b in range(4):
                    k = 4 * i + b
                    b2 = (b + 2) % 4
                    pltpu.make_async_copy(gh.at[srcb.at[k]], rbs[b], sgs[b]).wait()
                    pltpu.async_copy(rbs[b], acc.at[dstb.at[k]], sss[b], add=True)

                    @pl.when(k + 2 < ECH)
                    def _():
                        @pl.when(k >= 2)
                        def _():
                            pltpu.make_async_copy(
                                rbs[b2], acc.at[dstb.at[k - 2]], sss[b2]).wait()
                        pltpu.async_copy(gh.at[srcb.at[k + 2]], rbs[b2], sgs[b2])
                return carry2
            lax.fori_loop(0, ECH // 4, _edge, 0)
            for b in range(4):
                pltpu.make_async_copy(
                    rbs[b], acc.at[dstb.at[ECH - 4 + b]], sss[b]).wait()
            plsc.subcore_barrier()

            for j in range(5):
                r0 = base_r + 128 * j
                pltpu.sync_copy(acc.at[pl.ds(r0, 128)], rowA.at[pl.ds(0, 128)])
                pltpu.sync_copy(tbuf.at[c].at[h].at[p].at[pl.ds(r0, 128)], rowB.at[pl.ds(0, 128)])
                pltpu.sync_copy(low.at[c].at[h].at[pl.ds(r0, 128)], bufL.at[pl.ds(0, 128)])
                pltpu.sync_copy(high.at[c].at[h].at[pl.ds(r0, 128)], bufH.at[pl.ds(0, 128)])

                def _drow(w, carry2):
                    for g in range(DH // 16):
                        ix = pl.ds(16 * g, 16)
                        dv = dinvb[128 * j + w, ix]
                        pv = rowA[w, ix]
                        t2 = rowB[w, ix]
                        tn = (scale * dv) * pv - msub * t2
                        rowB[w, ix] = tn
                        rowA[w, ix] = dv * tn
                        bufL[w, ix] = bufL[w, ix] + cr * tn
                        bufH[w, ix] = bufH[w, ix] + (sgn * cr) * tn
                    return carry2
                lax.fori_loop(0, 128, _drow, 0)
                pltpu.sync_copy(rowB.at[pl.ds(0, 128)], tbuf.at[c].at[h].at[p].at[pl.ds(r0, 128)])
                pltpu.sync_copy(rowA.at[pl.ds(0, 128)], gbuf.at[c].at[h].at[pl.ds(r0, 128)])
                pltpu.sync_copy(bufL.at[pl.ds(0, 128)], low.at[c].at[h].at[pl.ds(r0, 128)])
                pltpu.sync_copy(bufH.at[pl.ds(0, 128)], high.at[c].at[h].at[pl.ds(r0, 128)])
                pltpu.sync_copy(zerob, acc.at[pl.ds(r0, 128)])
            plsc.subcore_barrier()
        return carry
    lax.fori_loop(1, KCH + 1, _round, 0)


_sc_cheb = functools.partial(
    pl.kernel,
    out_type=[
        jax.ShapeDtypeStruct((2, NH, ROWS, DH), jnp.float32),
        jax.ShapeDtypeStruct((2, NH, ROWS, DH), jnp.float32),
        jax.ShapeDtypeStruct((2, NH, 2, ROWS, DH), jnp.float32),
        jax.ShapeDtypeStruct((2, NH, ROWS, DH), jnp.float32),
    ],
    mesh=plsc.VectorSubcoreMesh(core_axis_name="c", subcore_axis_name="s"),
    compiler_params=pltpu.CompilerParams(use_tc_tiling_on_sc=False),
    scratch_types=[
        pltpu.VMEM((ECH, 256), jnp.int32),
        pltpu.VMEM((ECH, 256), jnp.int32),
        pltpu.VMEM((256, DH), jnp.float32),
        pltpu.VMEM((256, DH), jnp.float32),
        pltpu.VMEM((256, DH), jnp.float32),
        pltpu.VMEM((256, DH), jnp.float32),
        pltpu.VMEM((128, DH), jnp.float32),
        pltpu.VMEM((TROWS, DH), jnp.float32),
        pltpu.VMEM((32,), jnp.float32),
        pltpu.VMEM_SHARED((ROWS, DH), jnp.float32),
        pltpu.SemaphoreType.DMA,
        pltpu.SemaphoreType.DMA,
        pltpu.SemaphoreType.DMA,
        pltpu.SemaphoreType.DMA,
        pltpu.SemaphoreType.DMA,
        pltpu.SemaphoreType.DMA,
        pltpu.SemaphoreType.DMA,
        pltpu.SemaphoreType.DMA,
        pltpu.SemaphoreType.DMA,
    ],
)(_sc_body)


def _proj_body(x_ref, w_ref, z_ref):
    z_ref[0, 0] = lax.dot_general(x_ref[0], w_ref[...],
                                  (((1,), (1,)), ((), ())),
                                  precision=lax.Precision.HIGHEST,
                                  preferred_element_type=jnp.float32)


def _cat(ref, ci):
    return jnp.concatenate([ref[ci, h] for h in range(NH)], axis=-1)


def _b1_body(low_ref, high_ref, b1_ref, ab_ref, out_ref):
    i = pl.program_id(0)
    b = b1_ref[0]
    h1 = jnp.maximum(_cat(high_ref, 0) + b, 0.0)
    h2 = jnp.maximum(_cat(low_ref, 0) + b, 0.0)
    h = ab_ref[0, 0] * h1 + ab_ref[0, 1] * h2

    @pl.when(i == 0)
    def _():
        out_ref[...] = jnp.zeros_like(out_ref)
    out_ref[...] += jnp.sum(h, axis=0, keepdims=True)


def _b2_body(low_ref, high_ref, b1_ref, hsum_ref, wb_ref, bb_ref, out_ref):
    cm = jnp.maximum(hsum_ref[...] * jnp.float32(1.0 / N), 0.0)
    wrow = lax.dot_general(cm, wb_ref[0], (((1,), (1,)), ((), ())),
                           precision=lax.Precision.HIGHEST,
                           preferred_element_type=jnp.float32)
    b = b1_ref[0]
    bbv = bb_ref[0, 0]
    h2 = jnp.maximum(_cat(low_ref, 0) + b, 0.0)
    h1 = jnp.maximum(_cat(high_ref, 0) + b, 0.0)
    h4 = jnp.maximum(_cat(low_ref, 1) + b, 0.0)
    h3 = jnp.maximum(_cat(high_ref, 1) + b, 0.0)

    def dots(hm):
        r = lax.dot_general(hm, wrow, (((1,), (1,)), ((), ())),
                            precision=lax.Precision.HIGHEST,
                            preferred_element_type=jnp.float32)
        return r[:, 0] + bbv
    out_ref[0, :] = dots(h2)
    out_ref[1, :] = dots(h1)
    out_ref[2, :] = dots(h4)
    out_ref[3, :] = dots(h3)


def kernel(edge_index, feat, shuf_feat, temp, W1, b1, alpha, beta, Wb, bb):
    f32 = jnp.float32
    pad_rows = jnp.zeros((ROWS - N, D), f32)
    Xp = jnp.stack([jnp.concatenate([feat, pad_rows]),
                    jnp.concatenate([shuf_feat, pad_rows])])
    pad_e = jnp.full((ETOT - E,), DUMMY, jnp.int32)
    srcp = jnp.concatenate([edge_index[0], pad_e])
    dstp = jnp.concatenate([edge_index[1], pad_e])
    pkT = ((srcp << 14) | dstp).reshape(NS * ECH, 256)
    coe = (2.0 / (KCH + 1)) * (jnp.asarray(_INTERP_F32) @ jax.nn.relu(temp))
    coe16 = jnp.concatenate([coe.astype(f32), jnp.zeros((16 - (KCH + 1),), f32)])

    Z = pl.pallas_call(
        _proj_body,
        grid=(2, NH, ROWS // 256),
        in_specs=[pl.BlockSpec((1, 256, D), lambda i, h, j: (i, j, 0)),
                  pl.BlockSpec((DH, D), lambda i, h, j: (h, 0))],
        out_specs=pl.BlockSpec((1, 1, 256, DH), lambda i, h, j: (i, h, j, 0)),
        out_shape=jax.ShapeDtypeStruct((2, NH, ROWS, DH), f32),
    )(Xp, W1)

    low, high, _t, _g = _sc_cheb(pkT, Z, coe16)

    ab = jnp.stack([alpha.astype(f32), beta.astype(f32)]).reshape(1, 2)
    hsum = pl.pallas_call(
        _b1_body,
        grid=(25,),
        in_specs=[pl.BlockSpec((1, NH, 400, DH), lambda i: (0, 0, i, 0)),
                  pl.BlockSpec((1, NH, 400, DH), lambda i: (0, 0, i, 0)),
                  pl.BlockSpec((1, D), lambda i: (0, 0)),
                  pl.BlockSpec((1, 2), lambda i: (0, 0))],
        out_specs=pl.BlockSpec((1, D), lambda i: (0, 0)),
        out_shape=jax.ShapeDtypeStruct((1, D), f32),
    )(low, high, b1.reshape(1, D), ab)

    out4 = pl.pallas_call(
        _b2_body,
        grid=(ROWS // 512,),
        in_specs=[pl.BlockSpec((2, NH, 512, DH), lambda i: (0, 0, i, 0)),
                  pl.BlockSpec((2, NH, 512, DH), lambda i: (0, 0, i, 0)),
                  pl.BlockSpec((1, D), lambda i: (0, 0)),
                  pl.BlockSpec((1, D), lambda i: (0, 0)),
                  pl.BlockSpec((1, D, D), lambda i: (0, 0, 0)),
                  pl.BlockSpec((1, 1), lambda i: (0, 0))],
        out_specs=pl.BlockSpec((4, 512), lambda i: (0, i)),
        out_shape=jax.ShapeDtypeStruct((4, ROWS), f32),
    )(low, high, b1.reshape(1, D), hsum, Wb, bb.reshape(1, 1))

    return jnp.concatenate([out4[0, :N], out4[1, :N], out4[2, :N], out4[3, :N]])
